# baseline (speedup 1.0000x reference)
import functools

import jax
import jax.numpy as jnp
from jax.experimental import pallas as pl
from jax.experimental.pallas import tpu as pltpu
from jax.experimental.pallas import tpu_sc as plsc

_EPS = 1e-5
_N = 16384
_NB = 8
_K = 20
_NCLS = 50

_R_KNN = 128
_R_EDGE = 256
_R_AFF = 2048
_R_GLB = 512
_R_HEAD = 256

_NW = 32
_CHUNK = 128


def _fold(layer):
    s = layer["g"] / jnp.sqrt(1.0 + _EPS)
    wt = layer["W"].T * s[None, :]
    c = layer["b"] * s + layer["be"]
    return wt, c


_W_KNN = 2560
_W_MID = 6144
_WALIGN = 256


def _knn_extract(ptsr, sqc, bidr, i0, ptst_w, sq_w, bid_w, wb, width, out_ref):
    dot = jax.lax.dot_general(
        ptsr, ptst_w, (((1,), (0,)), ((), ())),
        preferred_element_type=jnp.float32)
    d = (sqc + sq_w) - 2.0 * dot
    rows = jax.lax.broadcasted_iota(jnp.int32, (_R_KNN, width), 0) + i0
    cols = jax.lax.broadcasted_iota(jnp.int32, (_R_KNN, width), 1) + wb
    bad = (bidr != bid_w) | (rows == cols)
    d = jnp.where(bad, jnp.inf, d)
    for j in range(_K):
        m = jnp.min(d, axis=1, keepdims=True)
        eq = d == m
        idx = jnp.min(jnp.where(eq, cols, _N), axis=1, keepdims=True)
        out_ref[:, j:j + 1] = idx
        d = jnp.where(eq, jnp.inf, d)


def _knn_body(ptsr_ref, ptst_ref, sq_ref, bid_ref, sqc_ref, bidc_ref, out_ref):
    i0 = pl.program_id(0) * _R_KNN
    bidr = bidc_ref[...]
    bmin = jnp.min(bidr)
    bmax = jnp.max(bidr)
    bid_all = bid_ref[...]
    lo = jnp.sum((bid_all < bmin).astype(jnp.int32))
    hi = jnp.sum((bid_all <= bmax).astype(jnp.int32))
    wb = jnp.minimum((lo // _WALIGN) * _WALIGN, _N - _W_KNN)
    wbm = jnp.minimum((lo // _WALIGN) * _WALIGN, _N - _W_MID)
    fits = hi <= wb + _W_KNN
    fits_mid = hi <= wbm + _W_MID
    ptsr = ptsr_ref[...]
    sqc = sqc_ref[...]

    @pl.when(fits)
    def _():
        _knn_extract(ptsr, sqc, bidr, i0,
                     ptst_ref[:, pl.ds(wb, _W_KNN)],
                     sq_ref[:, pl.ds(wb, _W_KNN)],
                     bid_ref[:, pl.ds(wb, _W_KNN)],
                     wb, _W_KNN, out_ref)

    @pl.when(jnp.logical_not(fits) & fits_mid)
    def _():
        _knn_extract(ptsr, sqc, bidr, i0,
                     ptst_ref[:, pl.ds(wbm, _W_MID)],
                     sq_ref[:, pl.ds(wbm, _W_MID)],
                     bid_ref[:, pl.ds(wbm, _W_MID)],
                     wbm, _W_MID, out_ref)

    @pl.when(jnp.logical_not(fits_mid))
    def _():
        _knn_extract(ptsr, sqc, bidr, i0, ptst_ref[...], sq_ref[...],
                     bid_ref[...], 0, _N, out_ref)


def _knn(pts, bid):
    ptsp = jnp.pad(pts, ((0, 0), (0, 5)))
    ptst = ptsp.T
    sq = jnp.sum(pts * pts, axis=-1)
    grid = _N // _R_KNN
    return pl.pallas_call(
        _knn_body,
        grid=(grid,),
        in_specs=[
            pl.BlockSpec((_R_KNN, 8), lambda i: (i, 0)),
            pl.BlockSpec((8, _N), lambda i: (0, 0)),
            pl.BlockSpec((1, _N), lambda i: (0, 0)),
            pl.BlockSpec((1, _N), lambda i: (0, 0)),
            pl.BlockSpec((_R_KNN, 1), lambda i: (i, 0)),
            pl.BlockSpec((_R_KNN, 1), lambda i: (i, 0)),
        ],
        out_specs=pl.BlockSpec((_R_KNN, _K), lambda i: (i, 0)),
        out_shape=jax.ShapeDtypeStruct((_N, _K), jnp.int32),
    )(ptsp, ptst, sq[None, :], bid[None, :], sq[:, None], bid[:, None])


def _affine2_body(x_ref, wa_ref, ba_ref, wb_ref, a_ref, b_ref):
    x = x_ref[...]
    a_ref[...] = jax.lax.dot_general(
        x, wa_ref[...], (((1,), (0,)), ((), ())),
        preferred_element_type=jnp.float32) + ba_ref[...]
    b_ref[...] = jax.lax.dot_general(
        x, wb_ref[...], (((1,), (0,)), ((), ())),
        preferred_element_type=jnp.float32)


def _affine2(x, wa, ba, wb):
    din = x.shape[1]
    dout = wa.shape[1]
    grid = _N // _R_AFF
    return pl.pallas_call(
        _affine2_body,
        grid=(grid,),
        in_specs=[
            pl.BlockSpec((_R_AFF, din), lambda i: (i, 0)),
            pl.BlockSpec((din, dout), lambda i: (0, 0)),
            pl.BlockSpec((1, dout), lambda i: (0, 0)),
            pl.BlockSpec((din, dout), lambda i: (0, 0)),
        ],
        out_specs=[
            pl.BlockSpec((_R_AFF, dout), lambda i: (i, 0)),
            pl.BlockSpec((_R_AFF, dout), lambda i: (i, 0)),
        ],
        out_shape=[
            jax.ShapeDtypeStruct((_N, dout), jnp.float32),
            jax.ShapeDtypeStruct((_N, dout), jnp.float32),
        ],
    )(x, wa, ba, wb)


def _sc_gather(table, idx3):
    nch = idx3.shape[1]
    d = table.shape[1]
    mesh = plsc.VectorSubcoreMesh(core_axis_name="c", subcore_axis_name="s")

    @functools.partial(
        pl.kernel, mesh=mesh,
        compiler_params=pltpu.CompilerParams(use_tc_tiling_on_sc=False),
        out_type=jax.ShapeDtypeStruct((_NW * nch * _CHUNK, d), jnp.float32),
        scratch_types=[
            pltpu.VMEM((nch, _CHUNK), jnp.int32),
            pltpu.VMEM((_CHUNK, d), jnp.float32),
            pltpu.VMEM((_CHUNK, d), jnp.float32),
            pltpu.SemaphoreType.DMA,
            pltpu.SemaphoreType.DMA,
            pltpu.SemaphoreType.DMA,
        ],
    )
    def k(table_hbm, idx_hbm, out_hbm, idx_v, buf0, buf1, sem0, sem1, semi):
        wid = jax.lax.axis_index("s") * 2 + jax.lax.axis_index("c")
        pltpu.async_copy(idx_hbm.at[wid], idx_v, semi).wait()
        bufs = (buf0, buf1)
        sems = (sem0, sem1)

        def start(c, b):
            pltpu.async_copy(table_hbm.at[idx_v.at[c]], bufs[b], sems[b])

        def finish(c, b):
            pltpu.make_async_copy(table_hbm.at[idx_v.at[c]], bufs[b],
                                  sems[b]).wait()
            base = (wid * nch + c) * _CHUNK
            pltpu.sync_copy(bufs[b], out_hbm.at[pl.ds(base, _CHUNK)])

        start(0, 0)

        def body(g, carry):
            c0 = g * 2
            start(c0 + 1, 1)
            finish(c0, 0)

            @pl.when(c0 + 2 < nch)
            def _():
                start(c0 + 2, 0)

            finish(c0 + 1, 1)
            return carry

        jax.lax.fori_loop(0, nch // 2, body, 0)

    return k(table, idx3)


def _edge_mm_body(a_ref, g_ref, w_ref, b_ref, out_ref):
    a = a_ref[...]
    acc = jnp.full((_R_EDGE, a.shape[1]), -jnp.inf, jnp.float32)
    w = w_ref[...]
    for j in range(_K):
        h1 = jnp.maximum(a + g_ref[j], 0.0)
        acc = jnp.maximum(acc, jax.lax.dot_general(
            h1, w, (((1,), (0,)), ((), ())),
            preferred_element_type=jnp.float32))
    out_ref[...] = jnp.maximum(acc + b_ref[...], 0.0)


def _edge_mm(a, g, w, b):
    dout = w.shape[1]
    grid = _N // _R_EDGE
    return pl.pallas_call(
        _edge_mm_body,
        grid=(grid,),
        in_specs=[
            pl.BlockSpec((_R_EDGE, a.shape[1]), lambda i: (i, 0)),
            pl.BlockSpec((_K, _R_EDGE, g.shape[2]), lambda i: (0, i, 0)),
            pl.BlockSpec(w.shape, lambda i: (0, 0)),
            pl.BlockSpec((1, dout), lambda i: (0, 0)),
        ],
        out_specs=pl.BlockSpec((_R_EDGE, dout), lambda i: (i, 0)),
        out_shape=jax.ShapeDtypeStruct((_N, dout), jnp.float32),
    )(a, g, w, b)


def _edge_max_body(a_ref, g_ref, out_ref):
    acc = g_ref[0]
    for j in range(1, _K):
        acc = jnp.maximum(acc, g_ref[j])
    out_ref[...] = jnp.maximum(a_ref[...] + acc, 0.0)


def _edge_max(a, g):
    dout = a.shape[1]
    grid = _N // _R_EDGE
    return pl.pallas_call(
        _edge_max_body,
        grid=(grid,),
        in_specs=[
            pl.BlockSpec((_R_EDGE, dout), lambda i: (i, 0)),
            pl.BlockSpec((_K, _R_EDGE, dout), lambda i: (0, i, 0)),
        ],
        out_specs=pl.BlockSpec((_R_EDGE, dout), lambda i: (i, 0)),
        out_shape=jax.ShapeDtypeStruct((_N, dout), jnp.float32),
    )(a, g)


def _glb_body(o1_ref, o2_ref, o3_ref, bidc_ref, wg_ref, bg_ref, wG_ref,
              glb_ref, G_ref):
    pid = pl.program_id(0)
    cats = jnp.concatenate([o1_ref[...], o2_ref[...], o3_ref[...]], axis=1)
    y = jax.lax.dot_general(
        cats, wg_ref[...], (((1,), (0,)), ((), ())),
        preferred_element_type=jnp.float32) + bg_ref[...]
    y = jnp.maximum(y, 0.0)

    @pl.when(pid == 0)
    def _():
        glb_ref[...] = jnp.full((_NB, y.shape[1]), -jnp.inf, jnp.float32)

    bid = bidc_ref[...]
    parts = []
    for s in range(_NB):
        mask = bid == s
        parts.append(jnp.max(jnp.where(mask, y, -jnp.inf), axis=0,
                             keepdims=True))
    glb_ref[...] = jnp.maximum(glb_ref[...], jnp.concatenate(parts, axis=0))

    @pl.when(pid == pl.num_programs(0) - 1)
    def _():
        g = glb_ref[...]
        g = jnp.where(g > -jnp.inf, g, 0.0)
        G_ref[...] = jax.lax.dot_general(
            g, wG_ref[...], (((1,), (0,)), ((), ())),
            preferred_element_type=jnp.float32)


def _glb(o1, o2, o3, bid, wg, bg, wG):
    grid = _N // _R_GLB
    return pl.pallas_call(
        _glb_body,
        grid=(grid,),
        in_specs=[
            pl.BlockSpec((_R_GLB, 64), lambda i: (i, 0)),
            pl.BlockSpec((_R_GLB, 64), lambda i: (i, 0)),
            pl.BlockSpec((_R_GLB, 64), lambda i: (i, 0)),
            pl.BlockSpec((_R_GLB, 1), lambda i: (i, 0)),
            pl.BlockSpec((192, 1024), lambda i: (0, 0)),
            pl.BlockSpec((1, 1024), lambda i: (0, 0)),
            pl.BlockSpec((1024, 512), lambda i: (0, 0)),
        ],
        out_specs=[
            pl.BlockSpec((_NB, 1024), lambda i: (0, 0)),
            pl.BlockSpec((_NB, 512), lambda i: (0, 0)),
        ],
        out_shape=[
            jax.ShapeDtypeStruct((_NB, 1024), jnp.float32),
            jax.ShapeDtypeStruct((_NB, 512), jnp.float32),
        ],
    )(o1, o2, o3, bid[:, None], wg, bg, wG)


def _head_body(o1_ref, o2_ref, o3_ref, bidc_ref, G_ref, wc1_ref, b1_ref,
               w2_ref, b2_ref, w3_ref, b3_ref, out_ref):
    cats = jnp.concatenate([o1_ref[...], o2_ref[...], o3_ref[...]], axis=1)
    cls = jax.lax.broadcasted_iota(jnp.int32, (_R_HEAD, _NB), 1)
    oh = (bidc_ref[...] == cls).astype(jnp.float32)
    gl = jax.lax.dot_general(
        oh, G_ref[...], (((1,), (0,)), ((), ())),
        preferred_element_type=jnp.float32)
    y1 = gl + jax.lax.dot_general(
        cats, wc1_ref[...], (((1,), (0,)), ((), ())),
        preferred_element_type=jnp.float32) + b1_ref[...]
    y1 = jnp.maximum(y1, 0.0)
    y2 = jnp.maximum(jax.lax.dot_general(
        y1, w2_ref[...], (((1,), (0,)), ((), ())),
        preferred_element_type=jnp.float32) + b2_ref[...], 0.0)
    z = jax.lax.dot_general(
        y2, w3_ref[...], (((1,), (0,)), ((), ())),
        preferred_element_type=jnp.float32) + b3_ref[...]
    lane = jax.lax.broadcasted_iota(jnp.int32, z.shape, 1)
    valid = lane < _NCLS
    m = jnp.max(jnp.where(valid, z, -jnp.inf), axis=1, keepdims=True)
    e = jnp.where(valid, jnp.exp(z - m), 0.0)
    lse = m + jnp.log(jnp.sum(e, axis=1, keepdims=True))
    out_ref[...] = (z - lse)[:, :_NCLS]


def _head(o1, o2, o3, bid, G, wc1, b1, w2, b2, w3, b3):
    grid = _N // _R_HEAD
    return pl.pallas_call(
        _head_body,
        grid=(grid,),
        in_specs=[
            pl.BlockSpec((_R_HEAD, 64), lambda i: (i, 0)),
            pl.BlockSpec((_R_HEAD, 64), lambda i: (i, 0)),
            pl.BlockSpec((_R_HEAD, 64), lambda i: (i, 0)),
            pl.BlockSpec((_R_HEAD, 1), lambda i: (i, 0)),
            pl.BlockSpec((_NB, 512), lambda i: (0, 0)),
            pl.BlockSpec((192, 512), lambda i: (0, 0)),
            pl.BlockSpec((1, 512), lambda i: (0, 0)),
            pl.BlockSpec((512, 256), lambda i: (0, 0)),
            pl.BlockSpec((1, 256), lambda i: (0, 0)),
            pl.BlockSpec((256, 64), lambda i: (0, 0)),
            pl.BlockSpec((1, 64), lambda i: (0, 0)),
        ],
        out_specs=pl.BlockSpec((_R_HEAD, _NCLS), lambda i: (i, 0)),
        out_shape=jax.ShapeDtypeStruct((_N, _NCLS), jnp.float32),
    )(o1, o2, o3, bid[:, None], G, wc1, b1, w2, b2, w3, b3)


def _edge_layer_mats(layer0, xdim):
    wt, c = _fold(layer0)
    wa = wt[:xdim] - wt[xdim:]
    wb = wt[xdim:]
    return wa, c[None, :], wb


def kernel(pts, batch_ids, params):
    bid = batch_ids.astype(jnp.int32)
    nbrs = _knn(pts, bid)
    idx3 = nbrs.T.reshape(_NW, -1, _CHUNK)

    wa1, ba1, wb1 = _edge_layer_mats(params["ec1"][0], 3)
    wa1 = jnp.pad(wa1, ((0, 5), (0, 0)))
    wb1 = jnp.pad(wb1, ((0, 5), (0, 0)))
    ptsp = jnp.pad(pts, ((0, 0), (0, 5)))
    a1, b1t = _affine2(ptsp, wa1, ba1, wb1)
    g1 = _sc_gather(b1t, idx3).reshape(_K, _N, 64)
    w12, c12 = _fold(params["ec1"][1])
    out1 = _edge_mm(a1, g1, w12, c12[None, :])

    wa2, ba2, wb2 = _edge_layer_mats(params["ec2"][0], 64)
    a2, b2t = _affine2(out1, wa2, ba2, wb2)
    g2 = _sc_gather(b2t, idx3).reshape(_K, _N, 64)
    w22, c22 = _fold(params["ec2"][1])
    out2 = _edge_mm(a2, g2, w22, c22[None, :])

    wa3, ba3, wb3 = _edge_layer_mats(params["ec3"][0], 64)
    a3, b3t = _affine2(out2, wa3, ba3, wb3)
    g3 = _sc_gather(b3t, idx3).reshape(_K, _N, 64)
    out3 = _edge_max(a3, g3)

    wg, cg = _fold(params["glb"][0])
    w1, c1 = _fold(params["fc1"][0])
    _, G = _glb(out1, out2, out3, bid, wg, cg[None, :], w1[:1024])

    w2, c2 = _fold(params["fc2"][0])
    w3 = params["fc3"]["W"].T
    w3p = jnp.pad(w3, ((0, 0), (0, 64 - _NCLS)))
    b3p = jnp.pad(params["fc3"]["b"], (0, 64 - _NCLS))
    return _head(out1, out2, out3, bid, G, w1[1024:], c1[None, :],
                 w2, c2[None, :], w3p, b3p[None, :])


# speedup vs baseline: 1.8677x; 1.8677x over previous
import functools

import jax
import jax.numpy as jnp
from jax.experimental import pallas as pl
from jax.experimental.pallas import tpu as pltpu
from jax.experimental.pallas import tpu_sc as plsc

_EPS = 1e-5
_N = 16384
_NB = 8
_K = 20
_NCLS = 50

_R_KNN = 128
_R_EDGE = 256
_R_AFF = 2048
_R_GLB = 512
_R_HEAD = 256

_NW = 32
_CHUNK = 128


def _fold(layer):
    s = layer["g"] / jnp.sqrt(1.0 + _EPS)
    wt = layer["W"].T * s[None, :]
    c = layer["b"] * s + layer["be"]
    return wt, c


_W_KNN = 4096
_WALIGN = 1024


def _knn_extract(ptsr, sqc, bidr, i0, ptst_w, sq_w, bid_w, wb, width, out_ref):
    dot = jax.lax.dot_general(
        ptsr, ptst_w, (((1,), (0,)), ((), ())),
        preferred_element_type=jnp.float32)
    d = (sqc + sq_w) - 2.0 * dot
    rows = jax.lax.broadcasted_iota(jnp.int32, (_R_KNN, width), 0) + i0
    cols = jax.lax.broadcasted_iota(jnp.int32, (_R_KNN, width), 1) + wb
    bad = (bidr != bid_w) | (rows == cols)
    d = jnp.where(bad, jnp.inf, d)
    for j in range(_K):
        m = jnp.min(d, axis=1, keepdims=True)
        eq = d == m
        idx = jnp.min(jnp.where(eq, cols, _N), axis=1, keepdims=True)
        out_ref[:, j:j + 1] = idx
        d = jnp.where(eq, jnp.inf, d)


def _knn_body(ptsr_ref, ptst_ref, sq_ref, bid_ref, sqc_ref, bidc_ref, out_ref):
    i0 = pl.program_id(0) * _R_KNN
    bidr = bidc_ref[...]
    bmin = jnp.min(bidr)
    bmax = jnp.max(bidr)
    bid_all = bid_ref[...]
    lo = jnp.sum((bid_all < bmin).astype(jnp.int32))
    hi = jnp.sum((bid_all <= bmax).astype(jnp.int32))
    wb = jnp.minimum((lo // _WALIGN) * _WALIGN, _N - _W_KNN)
    fits = hi <= wb + _W_KNN
    ptsr = ptsr_ref[...]
    sqc = sqc_ref[...]

    @pl.when(fits)
    def _():
        _knn_extract(ptsr, sqc, bidr, i0,
                     ptst_ref[:, pl.ds(wb, _W_KNN)],
                     sq_ref[:, pl.ds(wb, _W_KNN)],
                     bid_ref[:, pl.ds(wb, _W_KNN)],
                     wb, _W_KNN, out_ref)

    @pl.when(jnp.logical_not(fits))
    def _():
        _knn_extract(ptsr, sqc, bidr, i0, ptst_ref[...], sq_ref[...],
                     bid_ref[...], 0, _N, out_ref)


def _knn(pts, bid):
    ptsp = jnp.pad(pts, ((0, 0), (0, 5)))
    ptst = ptsp.T
    sq = jnp.sum(pts * pts, axis=-1)
    grid = _N // _R_KNN
    return pl.pallas_call(
        _knn_body,
        grid=(grid,),
        in_specs=[
            pl.BlockSpec((_R_KNN, 8), lambda i: (i, 0)),
            pl.BlockSpec((8, _N), lambda i: (0, 0)),
            pl.BlockSpec((1, _N), lambda i: (0, 0)),
            pl.BlockSpec((1, _N), lambda i: (0, 0)),
            pl.BlockSpec((_R_KNN, 1), lambda i: (i, 0)),
            pl.BlockSpec((_R_KNN, 1), lambda i: (i, 0)),
        ],
        out_specs=pl.BlockSpec((_R_KNN, _K), lambda i: (i, 0)),
        out_shape=jax.ShapeDtypeStruct((_N, _K), jnp.int32),
    )(ptsp, ptst, sq[None, :], bid[None, :], sq[:, None], bid[:, None])


def _affine2_body(x_ref, wa_ref, ba_ref, wb_ref, a_ref, b_ref):
    x = x_ref[...]
    a_ref[...] = jax.lax.dot_general(
        x, wa_ref[...], (((1,), (0,)), ((), ())),
        preferred_element_type=jnp.float32) + ba_ref[...]
    b_ref[...] = jax.lax.dot_general(
        x, wb_ref[...], (((1,), (0,)), ((), ())),
        preferred_element_type=jnp.float32)


def _affine2(x, wa, ba, wb):
    din = x.shape[1]
    dout = wa.shape[1]
    grid = _N // _R_AFF
    return pl.pallas_call(
        _affine2_body,
        grid=(grid,),
        in_specs=[
            pl.BlockSpec((_R_AFF, din), lambda i: (i, 0)),
            pl.BlockSpec((din, dout), lambda i: (0, 0)),
            pl.BlockSpec((1, dout), lambda i: (0, 0)),
            pl.BlockSpec((din, dout), lambda i: (0, 0)),
        ],
        out_specs=[
            pl.BlockSpec((_R_AFF, dout), lambda i: (i, 0)),
            pl.BlockSpec((_R_AFF, dout), lambda i: (i, 0)),
        ],
        out_shape=[
            jax.ShapeDtypeStruct((_N, dout), jnp.float32),
            jax.ShapeDtypeStruct((_N, dout), jnp.float32),
        ],
    )(x, wa, ba, wb)


def _sc_gather(table, idx3):
    nch = idx3.shape[1]
    d = table.shape[1]
    mesh = plsc.VectorSubcoreMesh(core_axis_name="c", subcore_axis_name="s")

    @functools.partial(
        pl.kernel, mesh=mesh,
        compiler_params=pltpu.CompilerParams(use_tc_tiling_on_sc=False),
        out_type=jax.ShapeDtypeStruct((_NW * nch * _CHUNK, d), jnp.float32),
        scratch_types=[
            pltpu.VMEM((nch, _CHUNK), jnp.int32),
            pltpu.VMEM((_CHUNK, d), jnp.float32),
            pltpu.VMEM((_CHUNK, d), jnp.float32),
            pltpu.SemaphoreType.DMA,
            pltpu.SemaphoreType.DMA,
            pltpu.SemaphoreType.DMA,
        ],
    )
    def k(table_hbm, idx_hbm, out_hbm, idx_v, buf0, buf1, sem0, sem1, semi):
        wid = jax.lax.axis_index("s") * 2 + jax.lax.axis_index("c")
        pltpu.async_copy(idx_hbm.at[wid], idx_v, semi).wait()
        bufs = (buf0, buf1)
        sems = (sem0, sem1)

        def start(c, b):
            pltpu.async_copy(table_hbm.at[idx_v.at[c]], bufs[b], sems[b])

        def finish(c, b):
            pltpu.make_async_copy(table_hbm.at[idx_v.at[c]], bufs[b],
                                  sems[b]).wait()
            base = (wid * nch + c) * _CHUNK
            pltpu.sync_copy(bufs[b], out_hbm.at[pl.ds(base, _CHUNK)])

        start(0, 0)

        def body(g, carry):
            c0 = g * 2
            start(c0 + 1, 1)
            finish(c0, 0)

            @pl.when(c0 + 2 < nch)
            def _():
                start(c0 + 2, 0)

            finish(c0 + 1, 1)
            return carry

        jax.lax.fori_loop(0, nch // 2, body, 0)

    return k(table, idx3)


def _edge_mm_body(a_ref, g_ref, w_ref, b_ref, out_ref):
    a = a_ref[...]
    acc = jnp.full((_R_EDGE, a.shape[1]), -jnp.inf, jnp.float32)
    w = w_ref[...]
    for j in range(_K):
        h1 = jnp.maximum(a + g_ref[j], 0.0)
        acc = jnp.maximum(acc, jax.lax.dot_general(
            h1, w, (((1,), (0,)), ((), ())),
            preferred_element_type=jnp.float32))
    out_ref[...] = jnp.maximum(acc + b_ref[...], 0.0)


def _edge_mm(a, g, w, b):
    dout = w.shape[1]
    grid = _N // _R_EDGE
    return pl.pallas_call(
        _edge_mm_body,
        grid=(grid,),
        in_specs=[
            pl.BlockSpec((_R_EDGE, a.shape[1]), lambda i: (i, 0)),
            pl.BlockSpec((_K, _R_EDGE, g.shape[2]), lambda i: (0, i, 0)),
            pl.BlockSpec(w.shape, lambda i: (0, 0)),
            pl.BlockSpec((1, dout), lambda i: (0, 0)),
        ],
        out_specs=pl.BlockSpec((_R_EDGE, dout), lambda i: (i, 0)),
        out_shape=jax.ShapeDtypeStruct((_N, dout), jnp.float32),
    )(a, g, w, b)


def _edge_max_body(a_ref, g_ref, out_ref):
    acc = g_ref[0]
    for j in range(1, _K):
        acc = jnp.maximum(acc, g_ref[j])
    out_ref[...] = jnp.maximum(a_ref[...] + acc, 0.0)


def _edge_max(a, g):
    dout = a.shape[1]
    grid = _N // _R_EDGE
    return pl.pallas_call(
        _edge_max_body,
        grid=(grid,),
        in_specs=[
            pl.BlockSpec((_R_EDGE, dout), lambda i: (i, 0)),
            pl.BlockSpec((_K, _R_EDGE, dout), lambda i: (0, i, 0)),
        ],
        out_specs=pl.BlockSpec((_R_EDGE, dout), lambda i: (i, 0)),
        out_shape=jax.ShapeDtypeStruct((_N, dout), jnp.float32),
    )(a, g)


def _glb_body(o1_ref, o2_ref, o3_ref, bidc_ref, wg_ref, bg_ref, wG_ref,
              glb_ref, G_ref):
    pid = pl.program_id(0)
    cats = jnp.concatenate([o1_ref[...], o2_ref[...], o3_ref[...]], axis=1)
    y = jax.lax.dot_general(
        cats, wg_ref[...], (((1,), (0,)), ((), ())),
        preferred_element_type=jnp.float32) + bg_ref[...]
    y = jnp.maximum(y, 0.0)

    @pl.when(pid == 0)
    def _():
        glb_ref[...] = jnp.full((_NB, y.shape[1]), -jnp.inf, jnp.float32)

    bid = bidc_ref[...]
    parts = []
    for s in range(_NB):
        mask = bid == s
        parts.append(jnp.max(jnp.where(mask, y, -jnp.inf), axis=0,
                             keepdims=True))
    glb_ref[...] = jnp.maximum(glb_ref[...], jnp.concatenate(parts, axis=0))

    @pl.when(pid == pl.num_programs(0) - 1)
    def _():
        g = glb_ref[...]
        g = jnp.where(g > -jnp.inf, g, 0.0)
        G_ref[...] = jax.lax.dot_general(
            g, wG_ref[...], (((1,), (0,)), ((), ())),
            preferred_element_type=jnp.float32)


def _glb(o1, o2, o3, bid, wg, bg, wG):
    grid = _N // _R_GLB
    return pl.pallas_call(
        _glb_body,
        grid=(grid,),
        in_specs=[
            pl.BlockSpec((_R_GLB, 64), lambda i: (i, 0)),
            pl.BlockSpec((_R_GLB, 64), lambda i: (i, 0)),
            pl.BlockSpec((_R_GLB, 64), lambda i: (i, 0)),
            pl.BlockSpec((_R_GLB, 1), lambda i: (i, 0)),
            pl.BlockSpec((192, 1024), lambda i: (0, 0)),
            pl.BlockSpec((1, 1024), lambda i: (0, 0)),
            pl.BlockSpec((1024, 512), lambda i: (0, 0)),
        ],
        out_specs=[
            pl.BlockSpec((_NB, 1024), lambda i: (0, 0)),
            pl.BlockSpec((_NB, 512), lambda i: (0, 0)),
        ],
        out_shape=[
            jax.ShapeDtypeStruct((_NB, 1024), jnp.float32),
            jax.ShapeDtypeStruct((_NB, 512), jnp.float32),
        ],
    )(o1, o2, o3, bid[:, None], wg, bg, wG)


def _head_body(o1_ref, o2_ref, o3_ref, bidc_ref, G_ref, wc1_ref, b1_ref,
               w2_ref, b2_ref, w3_ref, b3_ref, out_ref):
    cats = jnp.concatenate([o1_ref[...], o2_ref[...], o3_ref[...]], axis=1)
    cls = jax.lax.broadcasted_iota(jnp.int32, (_R_HEAD, _NB), 1)
    oh = (bidc_ref[...] == cls).astype(jnp.float32)
    gl = jax.lax.dot_general(
        oh, G_ref[...], (((1,), (0,)), ((), ())),
        preferred_element_type=jnp.float32)
    y1 = gl + jax.lax.dot_general(
        cats, wc1_ref[...], (((1,), (0,)), ((), ())),
        preferred_element_type=jnp.float32) + b1_ref[...]
    y1 = jnp.maximum(y1, 0.0)
    y2 = jnp.maximum(jax.lax.dot_general(
        y1, w2_ref[...], (((1,), (0,)), ((), ())),
        preferred_element_type=jnp.float32) + b2_ref[...], 0.0)
    z = jax.lax.dot_general(
        y2, w3_ref[...], (((1,), (0,)), ((), ())),
        preferred_element_type=jnp.float32) + b3_ref[...]
    lane = jax.lax.broadcasted_iota(jnp.int32, z.shape, 1)
    valid = lane < _NCLS
    m = jnp.max(jnp.where(valid, z, -jnp.inf), axis=1, keepdims=True)
    e = jnp.where(valid, jnp.exp(z - m), 0.0)
    lse = m + jnp.log(jnp.sum(e, axis=1, keepdims=True))
    out_ref[...] = (z - lse)[:, :_NCLS]


def _head(o1, o2, o3, bid, G, wc1, b1, w2, b2, w3, b3):
    grid = _N // _R_HEAD
    return pl.pallas_call(
        _head_body,
        grid=(grid,),
        in_specs=[
            pl.BlockSpec((_R_HEAD, 64), lambda i: (i, 0)),
            pl.BlockSpec((_R_HEAD, 64), lambda i: (i, 0)),
            pl.BlockSpec((_R_HEAD, 64), lambda i: (i, 0)),
            pl.BlockSpec((_R_HEAD, 1), lambda i: (i, 0)),
            pl.BlockSpec((_NB, 512), lambda i: (0, 0)),
            pl.BlockSpec((192, 512), lambda i: (0, 0)),
            pl.BlockSpec((1, 512), lambda i: (0, 0)),
            pl.BlockSpec((512, 256), lambda i: (0, 0)),
            pl.BlockSpec((1, 256), lambda i: (0, 0)),
            pl.BlockSpec((256, 64), lambda i: (0, 0)),
            pl.BlockSpec((1, 64), lambda i: (0, 0)),
        ],
        out_specs=pl.BlockSpec((_R_HEAD, _NCLS), lambda i: (i, 0)),
        out_shape=jax.ShapeDtypeStruct((_N, _NCLS), jnp.float32),
    )(o1, o2, o3, bid[:, None], G, wc1, b1, w2, b2, w3, b3)


def _edge_layer_mats(layer0, xdim):
    wt, c = _fold(layer0)
    wa = wt[:xdim] - wt[xdim:]
    wb = wt[xdim:]
    return wa, c[None, :], wb


def kernel(pts, batch_ids, params):
    bid = batch_ids.astype(jnp.int32)
    nbrs = _knn(pts, bid)
    idx3 = nbrs.T.reshape(_NW, -1, _CHUNK)

    wa1, ba1, wb1 = _edge_layer_mats(params["ec1"][0], 3)
    wa1 = jnp.pad(wa1, ((0, 5), (0, 0)))
    wb1 = jnp.pad(wb1, ((0, 5), (0, 0)))
    ptsp = jnp.pad(pts, ((0, 0), (0, 5)))
    a1, b1t = _affine2(ptsp, wa1, ba1, wb1)
    g1 = _sc_gather(b1t, idx3).reshape(_K, _N, 64)
    w12, c12 = _fold(params["ec1"][1])
    out1 = _edge_mm(a1, g1, w12, c12[None, :])

    wa2, ba2, wb2 = _edge_layer_mats(params["ec2"][0], 64)
    a2, b2t = _affine2(out1, wa2, ba2, wb2)
    g2 = _sc_gather(b2t, idx3).reshape(_K, _N, 64)
    w22, c22 = _fold(params["ec2"][1])
    out2 = _edge_mm(a2, g2, w22, c22[None, :])

    wa3, ba3, wb3 = _edge_layer_mats(params["ec3"][0], 64)
    a3, b3t = _affine2(out2, wa3, ba3, wb3)
    g3 = _sc_gather(b3t, idx3).reshape(_K, _N, 64)
    out3 = _edge_max(a3, g3)

    wg, cg = _fold(params["glb"][0])
    w1, c1 = _fold(params["fc1"][0])
    _, G = _glb(out1, out2, out3, bid, wg, cg[None, :], w1[:1024])

    w2, c2 = _fold(params["fc2"][0])
    w3 = params["fc3"]["W"].T
    w3p = jnp.pad(w3, ((0, 0), (0, 64 - _NCLS)))
    b3p = jnp.pad(params["fc3"]["b"], (0, 64 - _NCLS))
    return _head(out1, out2, out3, bid, G, w1[1024:], c1[None, :],
                 w2, c2[None, :], w3p, b3p[None, :])


# speedup vs baseline: 1.8820x; 1.0076x over previous
import functools

import jax
import jax.numpy as jnp
from jax.experimental import pallas as pl
from jax.experimental.pallas import tpu as pltpu
from jax.experimental.pallas import tpu_sc as plsc

_EPS = 1e-5
_N = 16384
_NB = 8
_K = 20
_NCLS = 50

_R_KNN = 128
_R_EDGE = 256
_R_AFF = 2048
_R_GLB = 512
_R_HEAD = 256

_NW = 32
_CHUNK = 128


def _fold(layer):
    s = layer["g"] / jnp.sqrt(1.0 + _EPS)
    wt = layer["W"].T * s[None, :]
    c = layer["b"] * s + layer["be"]
    return wt, c


_W_KNN = 4096
_WALIGN = 1024


def _knn_extract(ptsr, sqc, bidr, i0, ptst_w, sq_w, bid_w, wb, width, out_ref):
    dot = jax.lax.dot_general(
        ptsr, ptst_w, (((1,), (0,)), ((), ())),
        preferred_element_type=jnp.float32)
    d = (sqc + sq_w) - 2.0 * dot
    rows = jax.lax.broadcasted_iota(jnp.int32, (_R_KNN, width), 0) + i0
    cols = jax.lax.broadcasted_iota(jnp.int32, (_R_KNN, width), 1) + wb
    bad = (bidr != bid_w) | (rows == cols)
    d = jnp.where(bad, jnp.inf, d)
    for j in range(_K):
        m = jnp.min(d, axis=1, keepdims=True)
        eq = d == m
        idx = jnp.min(jnp.where(eq, cols, _N), axis=1, keepdims=True)
        out_ref[:, j:j + 1] = idx
        d = jnp.where(eq, jnp.inf, d)


def _knn_body(ptsr_ref, ptst_ref, sq_ref, bid_ref, sqc_ref, bidc_ref,
              wa_ref, ba_ref, wb_ref, out_ref, a_ref, b_ref):
    i0 = pl.program_id(0) * _R_KNN
    a_ref[...] = jax.lax.dot_general(
        ptsr_ref[...], wa_ref[...], (((1,), (0,)), ((), ())),
        preferred_element_type=jnp.float32) + ba_ref[...]
    b_ref[...] = jax.lax.dot_general(
        ptsr_ref[...], wb_ref[...], (((1,), (0,)), ((), ())),
        preferred_element_type=jnp.float32)
    bidr = bidc_ref[...]
    bmin = jnp.min(bidr)
    bmax = jnp.max(bidr)
    bid_all = bid_ref[...]
    lo = jnp.sum((bid_all < bmin).astype(jnp.int32))
    hi = jnp.sum((bid_all <= bmax).astype(jnp.int32))
    wb = jnp.minimum((lo // _WALIGN) * _WALIGN, _N - _W_KNN)
    fits = hi <= wb + _W_KNN
    ptsr = ptsr_ref[...]
    sqc = sqc_ref[...]

    @pl.when(fits)
    def _():
        _knn_extract(ptsr, sqc, bidr, i0,
                     ptst_ref[:, pl.ds(wb, _W_KNN)],
                     sq_ref[:, pl.ds(wb, _W_KNN)],
                     bid_ref[:, pl.ds(wb, _W_KNN)],
                     wb, _W_KNN, out_ref)

    @pl.when(jnp.logical_not(fits))
    def _():
        _knn_extract(ptsr, sqc, bidr, i0, ptst_ref[...], sq_ref[...],
                     bid_ref[...], 0, _N, out_ref)


def _knn(pts, bid, wa, ba, wb):
    ptsp = jnp.pad(pts, ((0, 0), (0, 5)))
    ptst = ptsp.T
    sq = jnp.sum(pts * pts, axis=-1)
    grid = _N // _R_KNN
    return pl.pallas_call(
        _knn_body,
        grid=(grid,),
        in_specs=[
            pl.BlockSpec((_R_KNN, 8), lambda i: (i, 0)),
            pl.BlockSpec((8, _N), lambda i: (0, 0)),
            pl.BlockSpec((1, _N), lambda i: (0, 0)),
            pl.BlockSpec((1, _N), lambda i: (0, 0)),
            pl.BlockSpec((_R_KNN, 1), lambda i: (i, 0)),
            pl.BlockSpec((_R_KNN, 1), lambda i: (i, 0)),
            pl.BlockSpec((8, 64), lambda i: (0, 0)),
            pl.BlockSpec((1, 64), lambda i: (0, 0)),
            pl.BlockSpec((8, 64), lambda i: (0, 0)),
        ],
        out_specs=[
            pl.BlockSpec((_R_KNN, _K), lambda i: (i, 0)),
            pl.BlockSpec((_R_KNN, 64), lambda i: (i, 0)),
            pl.BlockSpec((_R_KNN, 64), lambda i: (i, 0)),
        ],
        out_shape=[
            jax.ShapeDtypeStruct((_N, _K), jnp.int32),
            jax.ShapeDtypeStruct((_N, 64), jnp.float32),
            jax.ShapeDtypeStruct((_N, 64), jnp.float32),
        ],
    )(ptsp, ptst, sq[None, :], bid[None, :], sq[:, None], bid[:, None],
      wa, ba, wb)


def _sc_gather(table, idx3):
    nch = idx3.shape[1]
    d = table.shape[1]
    mesh = plsc.VectorSubcoreMesh(core_axis_name="c", subcore_axis_name="s")

    @functools.partial(
        pl.kernel, mesh=mesh,
        compiler_params=pltpu.CompilerParams(use_tc_tiling_on_sc=False),
        out_type=jax.ShapeDtypeStruct((_NW * nch * _CHUNK, d), jnp.float32),
        scratch_types=[
            pltpu.VMEM((nch, _CHUNK), jnp.int32),
            pltpu.VMEM((_CHUNK, d), jnp.float32),
            pltpu.VMEM((_CHUNK, d), jnp.float32),
            pltpu.SemaphoreType.DMA,
            pltpu.SemaphoreType.DMA,
            pltpu.SemaphoreType.DMA,
        ],
    )
    def k(table_hbm, idx_hbm, out_hbm, idx_v, buf0, buf1, sem0, sem1, semi):
        wid = jax.lax.axis_index("s") * 2 + jax.lax.axis_index("c")
        pltpu.async_copy(idx_hbm.at[wid], idx_v, semi).wait()
        bufs = (buf0, buf1)
        sems = (sem0, sem1)

        def start(c, b):
            pltpu.async_copy(table_hbm.at[idx_v.at[c]], bufs[b], sems[b])

        def finish(c, b):
            pltpu.make_async_copy(table_hbm.at[idx_v.at[c]], bufs[b],
                                  sems[b]).wait()
            base = (wid * nch + c) * _CHUNK
            pltpu.sync_copy(bufs[b], out_hbm.at[pl.ds(base, _CHUNK)])

        start(0, 0)

        def body(g, carry):
            c0 = g * 2
            start(c0 + 1, 1)
            finish(c0, 0)

            @pl.when(c0 + 2 < nch)
            def _():
                start(c0 + 2, 0)

            finish(c0 + 1, 1)
            return carry

        jax.lax.fori_loop(0, nch // 2, body, 0)

    return k(table, idx3)


def _edge_mm_body(a_ref, g_ref, w_ref, b_ref, wa_ref, ba_ref, wb_ref,
                  out_ref, an_ref, bn_ref):
    a = a_ref[...]
    acc = jnp.full((_R_EDGE, a.shape[1]), -jnp.inf, jnp.float32)
    w = w_ref[...]
    for j in range(_K):
        h1 = jnp.maximum(a + g_ref[j], 0.0)
        acc = jnp.maximum(acc, jax.lax.dot_general(
            h1, w, (((1,), (0,)), ((), ())),
            preferred_element_type=jnp.float32))
    out = jnp.maximum(acc + b_ref[...], 0.0)
    out_ref[...] = out
    an_ref[...] = jax.lax.dot_general(
        out, wa_ref[...], (((1,), (0,)), ((), ())),
        preferred_element_type=jnp.float32) + ba_ref[...]
    bn_ref[...] = jax.lax.dot_general(
        out, wb_ref[...], (((1,), (0,)), ((), ())),
        preferred_element_type=jnp.float32)


def _edge_mm(a, g, w, b, wa, ba, wb):
    dout = w.shape[1]
    grid = _N // _R_EDGE
    return pl.pallas_call(
        _edge_mm_body,
        grid=(grid,),
        in_specs=[
            pl.BlockSpec((_R_EDGE, a.shape[1]), lambda i: (i, 0)),
            pl.BlockSpec((_K, _R_EDGE, g.shape[2]), lambda i: (0, i, 0)),
            pl.BlockSpec(w.shape, lambda i: (0, 0)),
            pl.BlockSpec((1, dout), lambda i: (0, 0)),
            pl.BlockSpec((dout, 64), lambda i: (0, 0)),
            pl.BlockSpec((1, 64), lambda i: (0, 0)),
            pl.BlockSpec((dout, 64), lambda i: (0, 0)),
        ],
        out_specs=[
            pl.BlockSpec((_R_EDGE, dout), lambda i: (i, 0)),
            pl.BlockSpec((_R_EDGE, 64), lambda i: (i, 0)),
            pl.BlockSpec((_R_EDGE, 64), lambda i: (i, 0)),
        ],
        out_shape=[
            jax.ShapeDtypeStruct((_N, dout), jnp.float32),
            jax.ShapeDtypeStruct((_N, 64), jnp.float32),
            jax.ShapeDtypeStruct((_N, 64), jnp.float32),
        ],
    )(a, g, w, b, wa, ba, wb)


def _edge_max_body(a_ref, g_ref, out_ref):
    acc = g_ref[0]
    for j in range(1, _K):
        acc = jnp.maximum(acc, g_ref[j])
    out_ref[...] = jnp.maximum(a_ref[...] + acc, 0.0)


def _edge_max(a, g):
    dout = a.shape[1]
    grid = _N // _R_EDGE
    return pl.pallas_call(
        _edge_max_body,
        grid=(grid,),
        in_specs=[
            pl.BlockSpec((_R_EDGE, dout), lambda i: (i, 0)),
            pl.BlockSpec((_K, _R_EDGE, dout), lambda i: (0, i, 0)),
        ],
        out_specs=pl.BlockSpec((_R_EDGE, dout), lambda i: (i, 0)),
        out_shape=jax.ShapeDtypeStruct((_N, dout), jnp.float32),
    )(a, g)


def _glb_body(o1_ref, o2_ref, o3_ref, bidc_ref, wg_ref, bg_ref, wG_ref,
              glb_ref, G_ref):
    pid = pl.program_id(0)
    cats = jnp.concatenate([o1_ref[...], o2_ref[...], o3_ref[...]], axis=1)
    y = jax.lax.dot_general(
        cats, wg_ref[...], (((1,), (0,)), ((), ())),
        preferred_element_type=jnp.float32) + bg_ref[...]
    y = jnp.maximum(y, 0.0)

    @pl.when(pid == 0)
    def _():
        glb_ref[...] = jnp.full((_NB, y.shape[1]), -jnp.inf, jnp.float32)

    bid = bidc_ref[...]
    parts = []
    for s in range(_NB):
        mask = bid == s
        parts.append(jnp.max(jnp.where(mask, y, -jnp.inf), axis=0,
                             keepdims=True))
    glb_ref[...] = jnp.maximum(glb_ref[...], jnp.concatenate(parts, axis=0))

    @pl.when(pid == pl.num_programs(0) - 1)
    def _():
        g = glb_ref[...]
        g = jnp.where(g > -jnp.inf, g, 0.0)
        G_ref[...] = jax.lax.dot_general(
            g, wG_ref[...], (((1,), (0,)), ((), ())),
            preferred_element_type=jnp.float32)


def _glb(o1, o2, o3, bid, wg, bg, wG):
    grid = _N // _R_GLB
    return pl.pallas_call(
        _glb_body,
        grid=(grid,),
        in_specs=[
            pl.BlockSpec((_R_GLB, 64), lambda i: (i, 0)),
            pl.BlockSpec((_R_GLB, 64), lambda i: (i, 0)),
            pl.BlockSpec((_R_GLB, 64), lambda i: (i, 0)),
            pl.BlockSpec((_R_GLB, 1), lambda i: (i, 0)),
            pl.BlockSpec((192, 1024), lambda i: (0, 0)),
            pl.BlockSpec((1, 1024), lambda i: (0, 0)),
            pl.BlockSpec((1024, 512), lambda i: (0, 0)),
        ],
        out_specs=[
            pl.BlockSpec((_NB, 1024), lambda i: (0, 0)),
            pl.BlockSpec((_NB, 512), lambda i: (0, 0)),
        ],
        out_shape=[
            jax.ShapeDtypeStruct((_NB, 1024), jnp.float32),
            jax.ShapeDtypeStruct((_NB, 512), jnp.float32),
        ],
    )(o1, o2, o3, bid[:, None], wg, bg, wG)


def _head_body(o1_ref, o2_ref, o3_ref, bidc_ref, G_ref, wc1_ref, b1_ref,
               w2_ref, b2_ref, w3_ref, b3_ref, out_ref):
    cats = jnp.concatenate([o1_ref[...], o2_ref[...], o3_ref[...]], axis=1)
    cls = jax.lax.broadcasted_iota(jnp.int32, (_R_HEAD, _NB), 1)
    oh = (bidc_ref[...] == cls).astype(jnp.float32)
    gl = jax.lax.dot_general(
        oh, G_ref[...], (((1,), (0,)), ((), ())),
        preferred_element_type=jnp.float32)
    y1 = gl + jax.lax.dot_general(
        cats, wc1_ref[...], (((1,), (0,)), ((), ())),
        preferred_element_type=jnp.float32) + b1_ref[...]
    y1 = jnp.maximum(y1, 0.0)
    y2 = jnp.maximum(jax.lax.dot_general(
        y1, w2_ref[...], (((1,), (0,)), ((), ())),
        preferred_element_type=jnp.float32) + b2_ref[...], 0.0)
    z = jax.lax.dot_general(
        y2, w3_ref[...], (((1,), (0,)), ((), ())),
        preferred_element_type=jnp.float32) + b3_ref[...]
    lane = jax.lax.broadcasted_iota(jnp.int32, z.shape, 1)
    valid = lane < _NCLS
    m = jnp.max(jnp.where(valid, z, -jnp.inf), axis=1, keepdims=True)
    e = jnp.where(valid, jnp.exp(z - m), 0.0)
    lse = m + jnp.log(jnp.sum(e, axis=1, keepdims=True))
    out_ref[...] = (z - lse)[:, :_NCLS]


def _head(o1, o2, o3, bid, G, wc1, b1, w2, b2, w3, b3):
    grid = _N // _R_HEAD
    return pl.pallas_call(
        _head_body,
        grid=(grid,),
        in_specs=[
            pl.BlockSpec((_R_HEAD, 64), lambda i: (i, 0)),
            pl.BlockSpec((_R_HEAD, 64), lambda i: (i, 0)),
            pl.BlockSpec((_R_HEAD, 64), lambda i: (i, 0)),
            pl.BlockSpec((_R_HEAD, 1), lambda i: (i, 0)),
            pl.BlockSpec((_NB, 512), lambda i: (0, 0)),
            pl.BlockSpec((192, 512), lambda i: (0, 0)),
            pl.BlockSpec((1, 512), lambda i: (0, 0)),
            pl.BlockSpec((512, 256), lambda i: (0, 0)),
            pl.BlockSpec((1, 256), lambda i: (0, 0)),
            pl.BlockSpec((256, 64), lambda i: (0, 0)),
            pl.BlockSpec((1, 64), lambda i: (0, 0)),
        ],
        out_specs=pl.BlockSpec((_R_HEAD, _NCLS), lambda i: (i, 0)),
        out_shape=jax.ShapeDtypeStruct((_N, _NCLS), jnp.float32),
    )(o1, o2, o3, bid[:, None], G, wc1, b1, w2, b2, w3, b3)


def _edge_layer_mats(layer0, xdim):
    wt, c = _fold(layer0)
    wa = wt[:xdim] - wt[xdim:]
    wb = wt[xdim:]
    return wa, c[None, :], wb


def kernel(pts, batch_ids, params):
    bid = batch_ids.astype(jnp.int32)

    wa1, ba1, wb1 = _edge_layer_mats(params["ec1"][0], 3)
    wa1 = jnp.pad(wa1, ((0, 5), (0, 0)))
    wb1 = jnp.pad(wb1, ((0, 5), (0, 0)))
    nbrs, a1, b1t = _knn(pts, bid, wa1, ba1, wb1)
    idx3 = nbrs.T.reshape(_NW, -1, _CHUNK)
    g1 = _sc_gather(b1t, idx3).reshape(_K, _N, 64)
    w12, c12 = _fold(params["ec1"][1])
    wa2, ba2, wb2 = _edge_layer_mats(params["ec2"][0], 64)
    out1, a2, b2t = _edge_mm(a1, g1, w12, c12[None, :], wa2, ba2, wb2)

    g2 = _sc_gather(b2t, idx3).reshape(_K, _N, 64)
    w22, c22 = _fold(params["ec2"][1])
    wa3, ba3, wb3 = _edge_layer_mats(params["ec3"][0], 64)
    out2, a3, b3t = _edge_mm(a2, g2, w22, c22[None, :], wa3, ba3, wb3)

    g3 = _sc_gather(b3t, idx3).reshape(_K, _N, 64)
    out3 = _edge_max(a3, g3)

    wg, cg = _fold(params["glb"][0])
    w1, c1 = _fold(params["fc1"][0])
    _, G = _glb(out1, out2, out3, bid, wg, cg[None, :], w1[:1024])

    w2, c2 = _fold(params["fc2"][0])
    w3 = params["fc3"]["W"].T
    w3p = jnp.pad(w3, ((0, 0), (0, 64 - _NCLS)))
    b3p = jnp.pad(params["fc3"]["b"], (0, 64 - _NCLS))
    return _head(out1, out2, out3, bid, G, w1[1024:], c1[None, :],
                 w2, c2[None, :], w3p, b3p[None, :])
